# baseline (speedup 1.0000x reference)
import functools

import jax
import jax.numpy as jnp
from jax import lax
from jax.experimental import pallas as pl
from jax.experimental.pallas import tpu as pltpu
from jax.experimental.pallas import tpu_sc as plsc

E = 320000
DIM = 128
NC = 2
NS = 16
NW = NC * NS
ROWS_W = E // NW
C = 80
NCHUNK = ROWS_W // C
NPAIR = (NCHUNK - 1) // 2
NT = 60
NG = DIM // 16

_mesh = plsc.VectorSubcoreMesh(core_axis_name="c", subcore_axis_name="s")


@functools.partial(
    pl.kernel,
    mesh=_mesh,
    out_type=jax.ShapeDtypeStruct((E, DIM), jnp.float32),
    scratch_types=[
        pltpu.VMEM((C, DIM), jnp.float32),
        pltpu.VMEM((C, DIM), jnp.float32),
        pltpu.VMEM((C, DIM), jnp.float32),
        pltpu.VMEM((C, DIM), jnp.float32),
        pltpu.VMEM((C * 3,), jnp.int32),
        pltpu.VMEM((C * 3,), jnp.int32),
        pltpu.VMEM((NT, DIM), jnp.float32),
        pltpu.SemaphoreType.DMA,
        pltpu.SemaphoreType.DMA,
        pltpu.SemaphoreType.DMA,
        pltpu.SemaphoreType.DMA,
    ],
)
def _sc_bond(msg_hbm, attrs_hbm, tab_hbm, out_hbm,
             buf0, buf1, obuf0, obuf1, abuf0, abuf1, tab_v,
             semi0, semi1, semo0, semo1):
    wid = lax.axis_index("s") * NC + lax.axis_index("c")
    base = wid * ROWS_W
    pltpu.sync_copy(tab_hbm, tab_v)

    def start_load(k, b, a, sem):
        row0 = base + k * C
        pltpu.make_async_copy(msg_hbm.at[pl.ds(row0, C)], b, sem).start()
        pltpu.make_async_copy(attrs_hbm.at[pl.ds(row0 * 3, C * 3)], a, sem).start()

    def wait_load(b, a, sem):
        pltpu.make_async_copy(msg_hbm.at[pl.ds(0, C)], b, sem).wait()
        pltpu.make_async_copy(attrs_hbm.at[pl.ds(0, C * 3)], a, sem).wait()

    def start_store(k, o, sem):
        row0 = base + k * C
        pltpu.make_async_copy(o, out_hbm.at[pl.ds(row0, C)], sem).start()

    def wait_store(o, sem):
        pltpu.make_async_copy(o, out_hbm.at[pl.ds(0, C)], sem).wait()

    def compute(b, o, a):
        @plsc.parallel_loop(0, C // 16)
        def block_body(q):
            a0 = a[pl.ds(q * 48, 16)]
            a1 = a[pl.ds(q * 48 + 16, 16)]
            a2 = a[pl.ds(q * 48 + 32, 16)]
            regs = (a0, a1, a2)
            for j in range(16):
                lane = 3 * j
                w0 = regs[lane // 16][lane % 16]
                w1 = regs[(lane + 1) // 16][(lane + 1) % 16]
                w2 = regs[(lane + 2) // 16][(lane + 2) % 16]
                ci = w0 * 12 + w1 * 2 + w2
                r = q * 16 + j
                for g in range(NG):
                    sl = pl.ds(g * 16, 16)
                    o[r, sl] = jnp.maximum(b[r, sl] + tab_v[ci, sl], 0.0)

    start_load(0, buf0, abuf0, semi0)

    def pair_body(p, carry):
        k0 = 2 * p
        start_load(k0 + 1, buf1, abuf1, semi1)
        wait_load(buf0, abuf0, semi0)

        @pl.when(p > 0)
        def _():
            wait_store(obuf0, semo0)

        compute(buf0, obuf0, abuf0)
        start_store(k0, obuf0, semo0)

        start_load(k0 + 2, buf0, abuf0, semi0)
        wait_load(buf1, abuf1, semi1)

        @pl.when(p > 0)
        def _():
            wait_store(obuf1, semo1)

        compute(buf1, obuf1, abuf1)
        start_store(k0 + 1, obuf1, semo1)
        return carry

    lax.fori_loop(0, NPAIR, pair_body, 0)

    wait_load(buf0, abuf0, semi0)
    wait_store(obuf0, semo0)
    compute(buf0, obuf0, abuf0)
    start_store(NCHUNK - 1, obuf0, semo0)
    wait_store(obuf0, semo0)
    wait_store(obuf1, semo1)


def kernel(message, attrs, W0, W1, W2):
    ai = attrs.astype(jnp.int32).reshape(-1)
    tab = (W0[:, None, None, :] + W1[None, :, None, :]
           + W2[None, None, :, :]).reshape(NT, DIM)
    return _sc_bond(message, ai, tab)


# speedup vs baseline: 3.3564x; 3.3564x over previous
import functools

import jax
import jax.numpy as jnp
from jax import lax
from jax.experimental import pallas as pl
from jax.experimental.pallas import tpu as pltpu
from jax.experimental.pallas import tpu_sc as plsc

E = 320000
DIM = 128
NC = 2
NS = 16
NW = NC * NS
ROWS_W = E // NW
C = 200
NCHUNK = ROWS_W // C
NPAIR = NCHUNK // 2
NT = 60
NG = DIM // 16

_mesh = plsc.VectorSubcoreMesh(core_axis_name="c", subcore_axis_name="s")


@functools.partial(
    pl.kernel,
    mesh=_mesh,
    out_type=jax.ShapeDtypeStruct((E, DIM), jnp.float32),
    scratch_types=[
        pltpu.VMEM((C, DIM), jnp.float32),
        pltpu.VMEM((C, DIM), jnp.float32),
        pltpu.VMEM((C, DIM), jnp.float32),
        pltpu.VMEM((C, DIM), jnp.float32),
        pltpu.VMEM((C + 8,), jnp.int32),
        pltpu.VMEM((C + 8,), jnp.int32),
        pltpu.VMEM((NT, DIM), jnp.float32),
        pltpu.SemaphoreType.DMA,
        pltpu.SemaphoreType.DMA,
        pltpu.SemaphoreType.DMA,
        pltpu.SemaphoreType.DMA,
    ],
)
def _sc_bond(msg_hbm, attrs_hbm, tab_hbm, out_hbm,
             buf0, buf1, obuf0, obuf1, abuf0, abuf1, tab_v,
             semi0, semi1, semo0, semo1):
    wid = lax.axis_index("s") * NC + lax.axis_index("c")
    base = wid * ROWS_W
    pltpu.sync_copy(tab_hbm, tab_v)

    def start_load(k, b, a, sem):
        row0 = base + k * C
        pltpu.make_async_copy(msg_hbm.at[pl.ds(row0, C)], b, sem).start()
        pltpu.make_async_copy(attrs_hbm.at[pl.ds(row0, C)], a.at[pl.ds(0, C)],
                              sem).start()

    def wait_load(b, a, sem):
        pltpu.make_async_copy(msg_hbm.at[pl.ds(0, C)], b, sem).wait()
        pltpu.make_async_copy(attrs_hbm.at[pl.ds(0, C)], a.at[pl.ds(0, C)],
                              sem).wait()

    def start_store(k, o, sem):
        row0 = base + k * C
        pltpu.make_async_copy(o, out_hbm.at[pl.ds(row0, C)], sem).start()

    def wait_store(o, sem):
        pltpu.make_async_copy(o, out_hbm.at[pl.ds(0, C)], sem).wait()

    def compute(b, o, a):
        @plsc.parallel_loop(0, C // 8)
        def block_body(q):
            av = a[pl.ds(q * 8, 16)]
            for j in range(8):
                ci = av[j]
                r = q * 8 + j
                for g in range(NG):
                    sl = pl.ds(g * 16, 16)
                    o[r, sl] = jnp.maximum(b[r, sl] + tab_v[ci, sl], 0.0)

    start_load(0, buf0, abuf0, semi0)

    def pair_body(p, carry):
        k0 = 2 * p
        start_load(k0 + 1, buf1, abuf1, semi1)
        wait_load(buf0, abuf0, semi0)

        @pl.when(p > 0)
        def _():
            wait_store(obuf0, semo0)

        compute(buf0, obuf0, abuf0)
        start_store(k0, obuf0, semo0)

        @pl.when(p + 1 < NPAIR)
        def _():
            start_load(k0 + 2, buf0, abuf0, semi0)

        wait_load(buf1, abuf1, semi1)

        @pl.when(p > 0)
        def _():
            wait_store(obuf1, semo1)

        compute(buf1, obuf1, abuf1)
        start_store(k0 + 1, obuf1, semo1)
        return carry

    lax.fori_loop(0, NPAIR, pair_body, 0)
    wait_store(obuf0, semo0)
    wait_store(obuf1, semo1)


def kernel(message, attrs, W0, W1, W2):
    a32 = attrs.astype(jnp.int32)
    ci = a32[:, 0] * 12 + a32[:, 1] * 2 + a32[:, 2]
    tab = (W0[:, None, None, :] + W1[None, :, None, :]
           + W2[None, None, :, :]).reshape(NT, DIM)
    return _sc_bond(message, ci, tab)
